# baseline (device time: 40218 ns/iter reference)
import jax
import jax.numpy as jnp
from jax import lax
from jax.experimental import pallas as pl
from jax.experimental.pallas import tpu as pltpu

B, S, H, Dh, Dr = 2, 256, 16, 64, 32
D = 1024
DC_LOCAL = 64
SCALE = (Dh + Dr) ** -0.5
BF = jnp.bfloat16
F32 = jnp.float32


def _dot(a, b):
    return jnp.dot(a, b, preferred_element_type=F32)


def _dot_nt(a, b):
    return lax.dot_general(
        a, b, (((1,), (1,)), ((), ())), preferred_element_type=F32
    )


def kernel(x, Wdkv, Wuk, Wuv, Wq, Wqr, Wkr, Wo):
    def body(
        x_ref, wdkv_ref, wuk_ref, wuv_ref, wq_ref, wqr_ref, wkr_ref, wo_ref,
        out_ref, c_comm, wuk_comm, wuv_comm, o_buf, send_sems, recv_sems,
    ):
        my_x = lax.axis_index("x")
        my_y = lax.axis_index("y")
        my_z = lax.axis_index("z")
        partner = (1 - my_x, my_y, my_z)

        wdkv = wdkv_ref[...].astype(BF)
        xs = [x_ref[b].astype(BF) for b in range(B)]
        for b in range(B):
            c_comm[0, b] = _dot(xs[b], wdkv).astype(BF)
        wuk_comm[0] = wuk_ref[...].astype(BF)
        wuv_comm[0] = wuv_ref[...].astype(BF)

        barrier = pltpu.get_barrier_semaphore()
        pl.semaphore_signal(
            barrier, inc=1, device_id=partner,
            device_id_type=pl.DeviceIdType.MESH,
        )
        pl.semaphore_wait(barrier, 1)

        rdmas = []
        for i, buf in enumerate((c_comm, wuk_comm, wuv_comm)):
            rdma = pltpu.make_async_remote_copy(
                src_ref=buf.at[0],
                dst_ref=buf.at[1],
                send_sem=send_sems.at[i],
                recv_sem=recv_sems.at[i],
                device_id=partner,
                device_id_type=pl.DeviceIdType.MESH,
            )
            rdma.start()
            rdmas.append(rdma)

        wq = wq_ref[...].astype(BF)
        wqr = wqr_ref[...].astype(BF)
        wkr = wkr_ref[...].astype(BF)
        qs = [_dot(xs[b], wq).astype(BF) for b in range(B)]
        qrs = [_dot(xs[b], wqr).astype(BF) for b in range(B)]
        krs = [_dot(xs[b], wkr).astype(BF) for b in range(B)]

        for rdma in rdmas:
            rdma.wait()

        wo = wo_ref[...].astype(BF)
        for b in range(B):
            k_b = (
                _dot(c_comm[0, b], wuk_comm[0])
                + _dot(c_comm[1, b], wuk_comm[1])
            ).astype(BF)
            v_b = (
                _dot(c_comm[0, b], wuv_comm[0])
                + _dot(c_comm[1, b], wuv_comm[1])
            ).astype(BF)
            for h in range(H):
                hd = slice(h * Dh, (h + 1) * Dh)
                scores = (
                    _dot_nt(qs[b][:, hd], k_b[:, hd])
                    + _dot_nt(qrs[b][:, h * Dr:(h + 1) * Dr], krs[b])
                ) * SCALE
                m = jnp.max(scores, axis=1, keepdims=True)
                e = jnp.exp(scores - m)
                p = (e / jnp.sum(e, axis=1, keepdims=True)).astype(BF)
                o_buf[:, hd] = lax.dot_general(
                    p, v_b[:, hd], (((1,), (0,)), ((), ())),
                    preferred_element_type=F32,
                ).astype(BF)
            out_ref[b] = _dot(o_buf[...], wo)

    return pl.pallas_call(
        body,
        out_shape=jax.ShapeDtypeStruct((B, S, D), jnp.float32),
        in_specs=[pl.BlockSpec(memory_space=pltpu.VMEM)] * 8,
        out_specs=pl.BlockSpec(memory_space=pltpu.VMEM),
        scratch_shapes=[
            pltpu.VMEM((2, B, S, DC_LOCAL), BF),
            pltpu.VMEM((2, DC_LOCAL, D), BF),
            pltpu.VMEM((2, DC_LOCAL, D), BF),
            pltpu.VMEM((S, H * Dh), BF),
            pltpu.SemaphoreType.DMA((3,)),
            pltpu.SemaphoreType.DMA((3,)),
        ],
        compiler_params=pltpu.CompilerParams(collective_id=0),
    )(x, Wdkv, Wuk, Wuv, Wq, Wqr, Wkr, Wo)


# device time: 30348 ns/iter; 1.3252x vs baseline; 1.3252x over previous
import jax
import jax.numpy as jnp
from jax import lax
from jax.experimental import pallas as pl
from jax.experimental.pallas import tpu as pltpu

B, S, H, Dh, Dr = 2, 256, 16, 64, 32
D = 1024
DC_LOCAL = 64
DC = 2 * DC_LOCAL
SCALE = (Dh + Dr) ** -0.5
BF = jnp.bfloat16
F32 = jnp.float32


def _dot(a, b):
    return jnp.dot(a, b, preferred_element_type=F32)


def _dot_nt(a, b):
    return lax.dot_general(
        a, b, (((1,), (1,)), ((), ())), preferred_element_type=F32
    )


def kernel(x, Wdkv, Wuk, Wuv, Wq, Wqr, Wkr, Wo):
    def body(
        x_ref, wdkrt_ref, wukuv_ref, wq_ref, wqr_ref, wo_ref,
        out_ref, c_comm, wuk_comm, wuv_comm, send_sems, recv_sems,
    ):
        my_x = lax.axis_index("x")
        my_y = lax.axis_index("y")
        my_z = lax.axis_index("z")
        partner = (1 - my_x, my_y, my_z)

        wdkrt = wdkrt_ref[...].astype(BF)
        xs = [x_ref[b].astype(BF) for b in range(B)]
        krs = []
        for b in range(B):
            ckr = _dot_nt(xs[b], wdkrt)
            c_comm[0, b] = ckr[:, :DC_LOCAL].astype(BF)
            krs.append(ckr[:, DC_LOCAL:].astype(BF))
        wuk_comm[0] = wukuv_ref[:DC_LOCAL].astype(BF)
        wuv_comm[0] = wukuv_ref[DC_LOCAL:].astype(BF)

        barrier = pltpu.get_barrier_semaphore()
        pl.semaphore_signal(
            barrier, inc=1, device_id=partner,
            device_id_type=pl.DeviceIdType.MESH,
        )
        pl.semaphore_wait(barrier, 1)

        rdmas = []
        for i, buf in enumerate((c_comm, wuk_comm, wuv_comm)):
            rdma = pltpu.make_async_remote_copy(
                src_ref=buf.at[0],
                dst_ref=buf.at[1],
                send_sem=send_sems.at[i],
                recv_sem=recv_sems.at[i],
                device_id=partner,
                device_id_type=pl.DeviceIdType.MESH,
            )
            rdma.start()
            rdmas.append(rdma)

        wq = (wq_ref[...] * SCALE).astype(BF)
        wqr = (wqr_ref[...] * SCALE).astype(BF)
        qs = [_dot(xs[b], wq).astype(BF) for b in range(B)]
        qrs = [_dot(xs[b], wqr).astype(BF) for b in range(B)]
        wo = wo_ref[...].astype(BF)

        for rdma in rdmas:
            rdma.wait()

        wuk_full = wuk_comm[...].reshape(DC, D)
        wuv_full = wuv_comm[...].reshape(DC, D)
        for b in range(B):
            ccat = jnp.concatenate([c_comm[0, b], c_comm[1, b]], axis=1)
            k_b = _dot(ccat, wuk_full).astype(BF)
            v_b = _dot(ccat, wuv_full).astype(BF)
            o_heads = []
            for h in range(H):
                hd = slice(h * Dh, (h + 1) * Dh)
                e = jnp.exp(
                    _dot_nt(qs[b][:, hd], k_b[:, hd])
                    + _dot_nt(qrs[b][:, h * Dr:(h + 1) * Dr], krs[b])
                )
                rs = jnp.sum(e, axis=1, keepdims=True)
                o_h = lax.dot_general(
                    e.astype(BF), v_b[:, hd], (((1,), (0,)), ((), ())),
                    preferred_element_type=F32,
                )
                o_heads.append((o_h / rs).astype(BF))
            out_ref[b] = _dot(jnp.concatenate(o_heads, axis=1), wo)

    wdkr_t = jnp.concatenate([Wdkv, Wkr], axis=1).T
    wukuv = jnp.concatenate([Wuk, Wuv], axis=0)
    return pl.pallas_call(
        body,
        out_shape=jax.ShapeDtypeStruct((B, S, D), jnp.float32),
        in_specs=[pl.BlockSpec(memory_space=pltpu.VMEM)] * 6,
        out_specs=pl.BlockSpec(memory_space=pltpu.VMEM),
        scratch_shapes=[
            pltpu.VMEM((2, B, S, DC_LOCAL), BF),
            pltpu.VMEM((2, DC_LOCAL, D), BF),
            pltpu.VMEM((2, DC_LOCAL, D), BF),
            pltpu.SemaphoreType.DMA((3,)),
            pltpu.SemaphoreType.DMA((3,)),
        ],
        compiler_params=pltpu.CompilerParams(collective_id=0),
    )(x, wdkr_t, wukuv, Wq, Wqr, Wo)


# device time: 25305 ns/iter; 1.5893x vs baseline; 1.1993x over previous
import jax
import jax.numpy as jnp
from jax import lax
from jax.experimental import pallas as pl
from jax.experimental.pallas import tpu as pltpu

B, S, H, Dh, Dr = 2, 256, 16, 64, 32
D = 1024
DC_LOCAL = 64
DC = 2 * DC_LOCAL
SCALE = (Dh + Dr) ** -0.5
BF = jnp.bfloat16
F32 = jnp.float32


def _dot(a, b):
    return jnp.dot(a, b, preferred_element_type=F32)


def _dot_nt(a, b):
    return lax.dot_general(
        a, b, (((1,), (1,)), ((), ())), preferred_element_type=F32
    )


def kernel(x, Wdkv, Wuk, Wuv, Wq, Wqr, Wkr, Wo):
    def body(
        x_ref, wdkrt_ref, wukuv_ref, wq_ref, wqr_ref, wo_ref,
        out_ref, c_comm, wuk_comm, wuv_comm, send_sems, recv_sems,
    ):
        my_x = lax.axis_index("x")
        my_y = lax.axis_index("y")
        my_z = lax.axis_index("z")
        partner = (1 - my_x, my_y, my_z)

        wdkrt = wdkrt_ref[...].astype(BF)
        xs = [x_ref[b].astype(BF) for b in range(B)]
        krs = []
        for b in range(B):
            ckr = _dot_nt(xs[b], wdkrt)
            c_comm[0, b] = ckr[:, :DC_LOCAL].astype(BF)
            krs.append(ckr[:, DC_LOCAL:].astype(BF))
        wuk_comm[0] = wukuv_ref[:DC_LOCAL].astype(BF)
        wuv_comm[0] = wukuv_ref[DC_LOCAL:].astype(BF)

        barrier = pltpu.get_barrier_semaphore()
        pl.semaphore_signal(
            barrier, inc=1, device_id=partner,
            device_id_type=pl.DeviceIdType.MESH,
        )
        pl.semaphore_wait(barrier, 1)

        rdmas = []
        for i, buf in enumerate((c_comm, wuk_comm, wuv_comm)):
            rdma = pltpu.make_async_remote_copy(
                src_ref=buf.at[0],
                dst_ref=buf.at[1],
                send_sem=send_sems.at[i],
                recv_sem=recv_sems.at[i],
                device_id=partner,
                device_id_type=pl.DeviceIdType.MESH,
            )
            rdma.start()
            rdmas.append(rdma)

        wq = (wq_ref[...] * SCALE).astype(BF)
        wqr = (wqr_ref[...] * SCALE).astype(BF)
        qs = [_dot(xs[b], wq).astype(BF) for b in range(B)]
        qrs = [_dot(xs[b], wqr).astype(BF) for b in range(B)]
        wo = wo_ref[...].astype(BF)

        for rdma in rdmas:
            rdma.wait()

        wuk_full = wuk_comm[...].reshape(DC, D)
        wuv_full = wuv_comm[...].reshape(DC, D)
        for b in range(B):
            ccat = jnp.concatenate([c_comm[0, b], c_comm[1, b]], axis=1)
            k_b = _dot(ccat, wuk_full).astype(BF)
            v_b = _dot(ccat, wuv_full).astype(BF)
            o_heads = []
            for h in range(H):
                hd = slice(h * Dh, (h + 1) * Dh)
                e = jnp.exp(
                    _dot_nt(qs[b][:, hd], k_b[:, hd])
                    + _dot_nt(qrs[b][:, h * Dr:(h + 1) * Dr], krs[b])
                )
                rs = jnp.sum(e, axis=1, keepdims=True)
                o_h = lax.dot_general(
                    e.astype(BF), v_b[:, hd], (((1,), (0,)), ((), ())),
                    preferred_element_type=F32,
                )
                o_heads.append((o_h / rs).astype(BF))
            out_ref[b] = _dot(jnp.concatenate(o_heads, axis=1), wo)

    wdkr_t = jnp.concatenate([Wdkv, Wkr], axis=1).T
    wukuv = jnp.concatenate([Wuk, Wuv], axis=0)
    return pl.pallas_call(
        body,
        out_shape=jax.ShapeDtypeStruct((B, S, D), jnp.float32),
        in_specs=[pl.BlockSpec(memory_space=pltpu.VMEM)] * 6,
        out_specs=pl.BlockSpec(memory_space=pltpu.VMEM),
        scratch_shapes=[
            pltpu.VMEM((2, B, S, DC_LOCAL), BF),
            pltpu.VMEM((2, DC_LOCAL, D), BF),
            pltpu.VMEM((2, DC_LOCAL, D), BF),
            pltpu.SemaphoreType.DMA((3,)),
            pltpu.SemaphoreType.DMA((3,)),
        ],
        compiler_params=pltpu.CompilerParams(
            collective_id=0,
            allow_input_fusion=[False, True, True, False, False, False],
        ),
    )(x, wdkr_t, wukuv, Wq, Wqr, Wo)
